# Initial kernel scaffold; baseline (speedup 1.0000x reference)
import jax
import jax.numpy as jnp
from jax.experimental import pallas as pl


def kernel(x, edge_index, train_edge_id, W1, b1, W2, b2, W3, b3, g1, be1, eps1, W4, b4, g2, be2, eps2, Wl1, bl1, Wl2, bl2, Wf, bf):
    raise NotImplementedError("write your pallas kernel here")


# speedup vs baseline: 7.3323x; 7.3323x over previous
import functools

import jax
import jax.numpy as jnp
from jax import lax
from jax.experimental import pallas as pl
from jax.experimental.pallas import tpu as pltpu
from jax.experimental.pallas import tpu_sc as plsc

_N = 10000
_E = 320000
_D = 128
_TE = 65536
_C = 7

_NC = 2
_NS = 16
_NW = _NC * _NS
_K = 128
_NCHUNK = _E // _K
_NITER = 80
_ZR = 80
_NZBLK = _N // _ZR


def _seg_sum_sc(rows_in, sd):
    mesh = plsc.VectorSubcoreMesh(core_axis_name="c", subcore_axis_name="s")

    @functools.partial(
        pl.kernel,
        out_type=jax.ShapeDtypeStruct((_NC, _N, _D), jnp.float32),
        mesh=mesh,
        scratch_types=[
            pltpu.VMEM((2, _K), jnp.int32),
            pltpu.VMEM((2, _K), jnp.int32),
            pltpu.VMEM((2, _K), jnp.int32),
            pltpu.VMEM((2, _K), jnp.int32),
            pltpu.VMEM((_K, _D), jnp.float32),
            pltpu.VMEM((_K, _D), jnp.float32),
            pltpu.VMEM_SHARED((_N, _D), jnp.float32),
            pltpu.SemaphoreType.DMA,
            pltpu.SemaphoreType.DMA,
            pltpu.SemaphoreType.DMA,
            pltpu.SemaphoreType.DMA,
            pltpu.SemaphoreType.DMA,
            pltpu.SemaphoreType.DMA,
        ],
    )
    def kfn(x_hbm, sd_hbm, pp_hbm, sidx0, sidx1, sidx2, sidx3, rows0, rows1,
            agg_sh, si0, si1, si2, si3, sr0, sr1):
        c = lax.axis_index("c")
        s = lax.axis_index("s")
        wid = s * _NC + c

        sidx = (sidx0, sidx1, sidx2, sidx3)
        sis = (si0, si1, si2, si3)
        rows = (rows0, rows1)
        srs = (sr0, sr1)

        zrow = jnp.zeros((16,), jnp.float32)

        def zbody(r, carry):
            for q in range(8):
                rows0[r, pl.ds(q * 16, 16)] = zrow
            return carry

        lax.fori_loop(0, _K, zbody, 0)

        def zsc(k, carry):
            blk = s + _NS * k

            @pl.when(blk < _NZBLK)
            def _():
                pltpu.sync_copy(rows0.at[pl.ds(0, _ZR)],
                                agg_sh.at[pl.ds(blk * _ZR, _ZR)])
            return carry

        lax.fori_loop(0, (_NZBLK + _NS - 1) // _NS, zsc, 0)
        plsc.subcore_barrier()

        def ch_of(j):
            cr = wid + _NW * j
            return jnp.minimum(cr, _NCHUNK - 1), cr < _NCHUNK

        def fire_idx(j, b):
            ch, _ = ch_of(j)
            pltpu.async_copy(sd_hbm.at[ch], sidx[b], sis[b])

        def wait_idx(j, b):
            ch, _ = ch_of(j)
            pltpu.make_async_copy(sd_hbm.at[ch], sidx[b], sis[b]).wait()

        def fire_rows(ib, rb):
            pltpu.async_copy(x_hbm.at[sidx[ib].at[0]], rows[rb], srs[rb])

        def wait_rows(ib, rb):
            pltpu.make_async_copy(x_hbm.at[sidx[ib].at[0]], rows[rb], srs[rb]).wait()

        def scatter(j, ib, rb):
            _, ok = ch_of(j)

            @pl.when(ok)
            def _():
                pltpu.sync_copy(rows[rb], agg_sh.at[sidx[ib].at[1]], add=True)

        fire_idx(0, 0)
        fire_idx(1, 1)
        fire_idx(2, 2)
        fire_idx(3, 3)
        wait_idx(0, 0)
        fire_rows(0, 0)

        def quad(kk, carry):
            b = 4 * kk
            wait_idx(b + 1, 1)
            fire_rows(1, 1)
            wait_rows(0, 0)
            scatter(b, 0, 0)
            fire_idx(b + 4, 0)
            wait_idx(b + 2, 2)
            fire_rows(2, 0)
            wait_rows(1, 1)
            scatter(b + 1, 1, 1)
            fire_idx(b + 5, 1)
            wait_idx(b + 3, 3)
            fire_rows(3, 1)
            wait_rows(2, 0)
            scatter(b + 2, 2, 0)
            fire_idx(b + 6, 2)
            wait_rows(3, 1)
            scatter(b + 3, 3, 1)
            fire_idx(b + 7, 3)
            wait_idx(b + 4, 0)
            fire_rows(0, 0)
            return carry

        lax.fori_loop(0, _NITER // 4, quad, 0)

        wait_rows(0, 0)
        wait_idx(_NITER + 1, 1)
        wait_idx(_NITER + 2, 2)
        wait_idx(_NITER + 3, 3)

        plsc.subcore_barrier()

        def wout(k, carry):
            blk = s + _NS * k

            @pl.when(blk < _NZBLK)
            def _():
                r0 = blk * _ZR
                pltpu.sync_copy(agg_sh.at[pl.ds(r0, _ZR)], rows0.at[pl.ds(0, _ZR)])
                pltpu.sync_copy(rows0.at[pl.ds(0, _ZR)], pp_hbm.at[c, pl.ds(r0, _ZR)])
            return carry

        lax.fori_loop(0, (_NZBLK + _NS - 1) // _NS, wout, 0)

    return kfn(rows_in, sd)


_TCHUNK = _TE // _K
_TPW = _TCHUNK // _NW


def _readout_sc(z, src1, dst1, te2):
    mesh = plsc.VectorSubcoreMesh(core_axis_name="c", subcore_axis_name="s")

    @functools.partial(
        pl.kernel,
        out_type=jax.ShapeDtypeStruct((_TE, _D), jnp.float32),
        mesh=mesh,
        scratch_types=[
            pltpu.VMEM((_K,), jnp.int32),
            pltpu.VMEM((_K,), jnp.int32),
            pltpu.VMEM((_K,), jnp.int32),
            pltpu.VMEM((_K, _D), jnp.float32),
            pltpu.VMEM((_K, _D), jnp.float32),
            pltpu.SemaphoreType.DMA,
            pltpu.SemaphoreType.DMA,
            pltpu.SemaphoreType.DMA,
        ],
    )
    def kfn(z_hbm, src_hbm, dst_hbm, te_hbm, w_hbm, te_v, sv, dv, x1_v, x2_v,
            sp, s1, s2):
        c = lax.axis_index("c")
        s = lax.axis_index("s")
        wid = s * _NC + c

        def chunk(jc, carry):
            ch = wid + _NW * jc
            pltpu.sync_copy(te_hbm.at[ch], te_v)
            pltpu.async_copy(src_hbm.at[te_v], sv, sp)
            pltpu.async_copy(dst_hbm.at[te_v], dv, s1)
            pltpu.make_async_copy(src_hbm.at[te_v], sv, sp).wait()
            pltpu.make_async_copy(dst_hbm.at[te_v], dv, s1).wait()
            pltpu.async_copy(z_hbm.at[sv], x1_v, s2)
            pltpu.async_copy(z_hbm.at[dv], x2_v, sp)
            pltpu.make_async_copy(z_hbm.at[sv], x1_v, s2).wait()
            pltpu.make_async_copy(z_hbm.at[dv], x2_v, sp).wait()

            def mrow(r, cc):
                for q in range(8):
                    sl = pl.ds(q * 16, 16)
                    x1_v[r, sl] = x1_v[r, sl] * x2_v[r, sl]
                return cc

            lax.fori_loop(0, _K, mrow, 0)
            pltpu.sync_copy(x1_v, w_hbm.at[pl.ds(ch * _K, _K)])
            return carry

        lax.fori_loop(0, _TPW, chunk, 0)

    return kfn(z, src1, dst1, te2)


_R = 2000
_G = _N // _R


def _full(shape):
    return pl.BlockSpec(shape, lambda i: (0,) * len(shape))


def _rows(shape):
    return pl.BlockSpec(shape, lambda i: (i,) + (0,) * (len(shape) - 1))


def _mlp1_tc(x, p0, p1, W1, b1, W2, b2, W3, b3, eps1):
    def body(eps_ref, x_ref, p0_ref, p1_ref, w1_ref, b1_ref, w2_ref, b2_ref,
             w3_ref, b3_ref, h_ref, st_ref, accs, accq):
        i = pl.program_id(0)
        t = (1.0 + eps_ref[0]) * x_ref[...] + p0_ref[...] + p1_ref[...]
        h = jnp.maximum(jnp.dot(t, w1_ref[...], preferred_element_type=jnp.float32) + b1_ref[...], 0.0)
        h = jnp.maximum(jnp.dot(h, w2_ref[...], preferred_element_type=jnp.float32) + b2_ref[...], 0.0)
        h = jnp.maximum(jnp.dot(h, w3_ref[...], preferred_element_type=jnp.float32) + b3_ref[...], 0.0)
        h_ref[...] = h

        @pl.when(i == 0)
        def _():
            accs[...] = jnp.zeros_like(accs)
            accq[...] = jnp.zeros_like(accq)

        accs[...] += jnp.sum(h, axis=0, keepdims=True)
        accq[...] += jnp.sum(h * h, axis=0, keepdims=True)

        @pl.when(i == _G - 1)
        def _():
            st_ref[0:1, :] = accs[...]
            st_ref[1:2, :] = accq[...]

    return pl.pallas_call(
        body,
        grid=(_G,),
        in_specs=[
            pl.BlockSpec(memory_space=pltpu.SMEM),
            _rows((_R, _D)), _rows((_R, _D)), _rows((_R, _D)),
            _full((_D, _D)), _full((1, _D)),
            _full((_D, _D)), _full((1, _D)),
            _full((_D, _D)), _full((1, _D)),
        ],
        out_specs=[_rows((_R, _D)), _full((2, _D))],
        out_shape=[
            jax.ShapeDtypeStruct((_N, _D), jnp.float32),
            jax.ShapeDtypeStruct((2, _D), jnp.float32),
        ],
        scratch_shapes=[
            pltpu.VMEM((1, _D), jnp.float32),
            pltpu.VMEM((1, _D), jnp.float32),
        ],
    )(eps1, x, p0, p1, W1, b1, W2, b2, W3, b3)


def _bnorm_tc(hp, st, g, be):
    def body(hp_ref, st_ref, g_ref, be_ref, h_ref):
        m = st_ref[0:1, :] * (1.0 / _N)
        q = st_ref[1:2, :] * (1.0 / _N)
        rstd = lax.rsqrt(q - m * m + 1e-5)
        h_ref[...] = (hp_ref[...] - m) * rstd * g_ref[...] + be_ref[...]

    return pl.pallas_call(
        body,
        grid=(_G,),
        in_specs=[_rows((_R, _D)), _full((2, _D)), _full((1, _D)), _full((1, _D))],
        out_specs=_rows((_R, _D)),
        out_shape=jax.ShapeDtypeStruct((_N, _D), jnp.float32),
    )(hp, st, g, be)


def _conv2_tc(h, q0, q1, W4, b4, eps2):
    def body(eps_ref, h_ref, q0_ref, q1_ref, w4_ref, b4_ref, o_ref, st_ref,
             accs, accq):
        i = pl.program_id(0)
        t = (1.0 + eps_ref[0]) * h_ref[...] + q0_ref[...] + q1_ref[...]
        o = jnp.maximum(jnp.dot(t, w4_ref[...], preferred_element_type=jnp.float32) + b4_ref[...], 0.0)
        o_ref[...] = o

        @pl.when(i == 0)
        def _():
            accs[...] = jnp.zeros_like(accs)
            accq[...] = jnp.zeros_like(accq)

        accs[...] += jnp.sum(o, axis=0, keepdims=True)
        accq[...] += jnp.sum(o * o, axis=0, keepdims=True)

        @pl.when(i == _G - 1)
        def _():
            st_ref[0:1, :] = accs[...]
            st_ref[1:2, :] = accq[...]

    return pl.pallas_call(
        body,
        grid=(_G,),
        in_specs=[
            pl.BlockSpec(memory_space=pltpu.SMEM),
            _rows((_R, _D)), _rows((_R, _D)), _rows((_R, _D)),
            _full((_D, _D)), _full((1, _D)),
        ],
        out_specs=[_rows((_R, _D)), _full((2, _D))],
        out_shape=[
            jax.ShapeDtypeStruct((_N, _D), jnp.float32),
            jax.ShapeDtypeStruct((2, _D), jnp.float32),
        ],
        scratch_shapes=[
            pltpu.VMEM((1, _D), jnp.float32),
            pltpu.VMEM((1, _D), jnp.float32),
        ],
    )(eps2, h, q0, q1, W4, b4)


def _head_tc(hp2, st2, g2, be2, Wl1, bl1, Wl2, bl2):
    def body(hp_ref, st_ref, g_ref, be_ref, wl1_ref, bl1_ref, wl2_ref,
             bl2_ref, z_ref):
        m = st_ref[0:1, :] * (1.0 / _N)
        q = st_ref[1:2, :] * (1.0 / _N)
        rstd = lax.rsqrt(q - m * m + 1e-5)
        hn = (hp_ref[...] - m) * rstd * g_ref[...] + be_ref[...]
        z1 = jnp.maximum(jnp.dot(hn, wl1_ref[...], preferred_element_type=jnp.float32) + bl1_ref[...], 0.0)
        z_ref[...] = jnp.dot(z1, wl2_ref[...], preferred_element_type=jnp.float32) + bl2_ref[...]

    return pl.pallas_call(
        body,
        grid=(_G,),
        in_specs=[
            _rows((_R, _D)), _full((2, _D)), _full((1, _D)), _full((1, _D)),
            _full((_D, _D)), _full((1, _D)), _full((_D, _D)), _full((1, _D)),
        ],
        out_specs=_rows((_R, _D)),
        out_shape=jax.ShapeDtypeStruct((_N, _D), jnp.float32),
    )(hp2, st2, g2, be2, Wl1, bl1, Wl2, bl2)


_RF = 2048
_GF = _TE // _RF


def _fc_tc(w, Wf, bf):
    def body(w_ref, wf_ref, bf_ref, o_ref):
        o_ref[...] = jnp.dot(w_ref[...], wf_ref[...], preferred_element_type=jnp.float32) + bf_ref[...]

    return pl.pallas_call(
        body,
        grid=(_GF,),
        in_specs=[
            pl.BlockSpec((_RF, _D), lambda i: (i, 0)),
            pl.BlockSpec((_D, _C), lambda i: (0, 0)),
            pl.BlockSpec((1, _C), lambda i: (0, 0)),
        ],
        out_specs=pl.BlockSpec((_RF, _C), lambda i: (i, 0)),
        out_shape=jax.ShapeDtypeStruct((_TE, _C), jnp.float32),
    )(w, Wf, bf)


def kernel(x, edge_index, train_edge_id, W1, b1, W2, b2, W3, b3, g1, be1, eps1,
           W4, b4, g2, be2, eps2, Wl1, bl1, Wl2, bl2, Wf, bf):
    src2 = edge_index[0].reshape(_NCHUNK, _K)
    dst2 = edge_index[1].reshape(_NCHUNK, _K)
    sd = jnp.stack([src2, dst2], axis=1)
    te2 = train_edge_id.reshape(_TE // _K, _K)

    b1r, b2r, b3r, b4r = (v.reshape(1, _D) for v in (b1, b2, b3, b4))
    bl1r, bl2r = bl1.reshape(1, _D), bl2.reshape(1, _D)
    g1r, be1r = g1.reshape(1, _D), be1.reshape(1, _D)
    g2r, be2r = g2.reshape(1, _D), be2.reshape(1, _D)
    bfr = bf.reshape(1, _C)
    e1 = eps1.reshape(1)
    e2 = eps2.reshape(1)

    pp = _seg_sum_sc(x, sd)
    hp1, st1 = _mlp1_tc(x, pp[0], pp[1], W1, b1r, W2, b2r, W3, b3r, e1)
    h = _bnorm_tc(hp1, st1, g1r, be1r)
    qq = _seg_sum_sc(h, sd)
    hp2, st2 = _conv2_tc(h, qq[0], qq[1], W4, b4r, e2)
    z = _head_tc(hp2, st2, g2r, be2r, Wl1, bl1r, Wl2, bl2r)
    w = _readout_sc(z, edge_index[0], edge_index[1], te2)
    return _fc_tc(w, Wf, bfr)


# speedup vs baseline: 7.5379x; 1.0280x over previous
import functools

import jax
import jax.numpy as jnp
from jax import lax
from jax.experimental import pallas as pl
from jax.experimental.pallas import tpu as pltpu
from jax.experimental.pallas import tpu_sc as plsc

_N = 10000
_E = 320000
_D = 128
_TE = 65536
_C = 7

_NC = 2
_NS = 16
_NW = _NC * _NS
_K = 128
_NCHUNK = _E // _K
_NITER = 80
_ZR = 80
_NZBLK = _N // _ZR


def _seg_sum_sc(rows_in, sd):
    mesh = plsc.VectorSubcoreMesh(core_axis_name="c", subcore_axis_name="s")

    @functools.partial(
        pl.kernel,
        out_type=jax.ShapeDtypeStruct((_NC, _N, _D), jnp.float32),
        mesh=mesh,
        scratch_types=[
            pltpu.VMEM((2, _K), jnp.int32),
            pltpu.VMEM((2, _K), jnp.int32),
            pltpu.VMEM((2, _K), jnp.int32),
            pltpu.VMEM((2, _K), jnp.int32),
            pltpu.VMEM((_K, _D), jnp.float32),
            pltpu.VMEM((_K, _D), jnp.float32),
            pltpu.VMEM_SHARED((_N, _D), jnp.float32),
            pltpu.SemaphoreType.DMA,
            pltpu.SemaphoreType.DMA,
            pltpu.SemaphoreType.DMA,
            pltpu.SemaphoreType.DMA,
            pltpu.SemaphoreType.DMA,
            pltpu.SemaphoreType.DMA,
        ],
    )
    def kfn(x_hbm, sd_hbm, pp_hbm, sidx0, sidx1, sidx2, sidx3, rows0, rows1,
            agg_sh, si0, si1, si2, si3, sr0, sr1):
        c = lax.axis_index("c")
        s = lax.axis_index("s")
        wid = s * _NC + c

        sidx = (sidx0, sidx1, sidx2, sidx3)
        sis = (si0, si1, si2, si3)
        rows = (rows0, rows1)
        srs = (sr0, sr1)

        zrow = jnp.zeros((16,), jnp.float32)

        def zbody(r, carry):
            for q in range(8):
                rows0[r, pl.ds(q * 16, 16)] = zrow
            return carry

        lax.fori_loop(0, _K, zbody, 0)

        def zsc(k, carry):
            blk = s + _NS * k

            @pl.when(blk < _NZBLK)
            def _():
                pltpu.sync_copy(rows0.at[pl.ds(0, _ZR)],
                                agg_sh.at[pl.ds(blk * _ZR, _ZR)])
            return carry

        lax.fori_loop(0, (_NZBLK + _NS - 1) // _NS, zsc, 0)
        plsc.subcore_barrier()

        def ch_of(j):
            cr = wid + _NW * j
            return jnp.minimum(cr, _NCHUNK - 1), cr < _NCHUNK

        def fire_idx(j, b):
            ch, _ = ch_of(j)
            pltpu.async_copy(sd_hbm.at[ch], sidx[b], sis[b])

        def wait_idx(j, b):
            ch, _ = ch_of(j)
            pltpu.make_async_copy(sd_hbm.at[ch], sidx[b], sis[b]).wait()

        def fire_rows(ib, rb):
            pltpu.async_copy(x_hbm.at[sidx[ib].at[0]], rows[rb], srs[rb])

        def wait_rows(ib, rb):
            pltpu.make_async_copy(x_hbm.at[sidx[ib].at[0]], rows[rb], srs[rb]).wait()

        def scatter(j, ib, rb):
            _, ok = ch_of(j)

            @pl.when(ok)
            def _():
                pltpu.sync_copy(rows[rb], agg_sh.at[sidx[ib].at[1]], add=True)

        fire_idx(0, 0)
        fire_idx(1, 1)
        fire_idx(2, 2)
        fire_idx(3, 3)
        wait_idx(0, 0)
        fire_rows(0, 0)

        def quad(kk, carry):
            b = 4 * kk
            wait_idx(b + 1, 1)
            fire_rows(1, 1)
            wait_rows(0, 0)
            scatter(b, 0, 0)
            fire_idx(b + 4, 0)
            wait_idx(b + 2, 2)
            fire_rows(2, 0)
            wait_rows(1, 1)
            scatter(b + 1, 1, 1)
            fire_idx(b + 5, 1)
            wait_idx(b + 3, 3)
            fire_rows(3, 1)
            wait_rows(2, 0)
            scatter(b + 2, 2, 0)
            fire_idx(b + 6, 2)
            wait_rows(3, 1)
            scatter(b + 3, 3, 1)
            fire_idx(b + 7, 3)
            wait_idx(b + 4, 0)
            fire_rows(0, 0)
            return carry

        lax.fori_loop(0, _NITER // 4, quad, 0)

        wait_rows(0, 0)
        wait_idx(_NITER + 1, 1)
        wait_idx(_NITER + 2, 2)
        wait_idx(_NITER + 3, 3)

        plsc.subcore_barrier()

        def wout(k, carry):
            blk = s + _NS * k

            @pl.when(blk < _NZBLK)
            def _():
                r0 = blk * _ZR
                pltpu.sync_copy(agg_sh.at[pl.ds(r0, _ZR)], rows0.at[pl.ds(0, _ZR)])
                pltpu.sync_copy(rows0.at[pl.ds(0, _ZR)], pp_hbm.at[c, pl.ds(r0, _ZR)])
            return carry

        lax.fori_loop(0, (_NZBLK + _NS - 1) // _NS, wout, 0)

    return kfn(rows_in, sd)


_TCHUNK = _TE // _K
_TPW = _TCHUNK // _NW


def _readout_sc(z, src1, dst1, te2):
    mesh = plsc.VectorSubcoreMesh(core_axis_name="c", subcore_axis_name="s")

    @functools.partial(
        pl.kernel,
        out_type=jax.ShapeDtypeStruct((_TE, _D), jnp.float32),
        mesh=mesh,
        scratch_types=[
            pltpu.VMEM((_K,), jnp.int32),
            pltpu.VMEM((_K,), jnp.int32),
            pltpu.VMEM((_K,), jnp.int32),
            pltpu.VMEM((_K, _D), jnp.float32),
            pltpu.VMEM((_K, _D), jnp.float32),
            pltpu.SemaphoreType.DMA,
            pltpu.SemaphoreType.DMA,
            pltpu.SemaphoreType.DMA,
        ],
    )
    def kfn(z_hbm, src_hbm, dst_hbm, te_hbm, w_hbm, te_v, sv, dv, x1_v, x2_v,
            sp, s1, s2):
        c = lax.axis_index("c")
        s = lax.axis_index("s")
        wid = s * _NC + c

        def chunk(jc, carry):
            ch = wid + _NW * jc
            pltpu.sync_copy(te_hbm.at[ch], te_v)
            pltpu.async_copy(src_hbm.at[te_v], sv, sp)
            pltpu.async_copy(dst_hbm.at[te_v], dv, s1)
            pltpu.make_async_copy(src_hbm.at[te_v], sv, sp).wait()
            pltpu.make_async_copy(dst_hbm.at[te_v], dv, s1).wait()
            pltpu.async_copy(z_hbm.at[sv], x1_v, s2)
            pltpu.async_copy(z_hbm.at[dv], x2_v, sp)
            pltpu.make_async_copy(z_hbm.at[sv], x1_v, s2).wait()
            pltpu.make_async_copy(z_hbm.at[dv], x2_v, sp).wait()

            def mrow(r, cc):
                for q in range(8):
                    sl = pl.ds(q * 16, 16)
                    x1_v[r, sl] = x1_v[r, sl] * x2_v[r, sl]
                return cc

            lax.fori_loop(0, _K, mrow, 0)
            pltpu.sync_copy(x1_v, w_hbm.at[pl.ds(ch * _K, _K)])
            return carry

        lax.fori_loop(0, _TPW, chunk, 0)

    return kfn(z, src1, dst1, te2)


def _vm(shape):
    return pl.BlockSpec(shape, lambda: (0,) * len(shape))


def _mlp1_tc(x, p0, p1, W1, b1, W2, b2, W3, b3, g1, be1, eps1):
    def body(eps_ref, x_ref, p0_ref, p1_ref, w1_ref, b1_ref, w2_ref, b2_ref,
             w3_ref, b3_ref, g_ref, be_ref, h_ref):
        t = (1.0 + eps_ref[0]) * x_ref[...] + p0_ref[...] + p1_ref[...]
        h = jnp.maximum(jnp.dot(t, w1_ref[...], preferred_element_type=jnp.float32) + b1_ref[...], 0.0)
        h = jnp.maximum(jnp.dot(h, w2_ref[...], preferred_element_type=jnp.float32) + b2_ref[...], 0.0)
        h = jnp.maximum(jnp.dot(h, w3_ref[...], preferred_element_type=jnp.float32) + b3_ref[...], 0.0)
        m = jnp.sum(h, axis=0, keepdims=True) * (1.0 / _N)
        q = jnp.sum(h * h, axis=0, keepdims=True) * (1.0 / _N)
        rstd = lax.rsqrt(q - m * m + 1e-5)
        h_ref[...] = (h - m) * rstd * g_ref[...] + be_ref[...]

    return pl.pallas_call(
        body,
        in_specs=[
            pl.BlockSpec(memory_space=pltpu.SMEM),
            _vm((_N, _D)), _vm((_N, _D)), _vm((_N, _D)),
            _vm((_D, _D)), _vm((1, _D)),
            _vm((_D, _D)), _vm((1, _D)),
            _vm((_D, _D)), _vm((1, _D)),
            _vm((1, _D)), _vm((1, _D)),
        ],
        out_specs=_vm((_N, _D)),
        out_shape=jax.ShapeDtypeStruct((_N, _D), jnp.float32),
    )(eps1, x, p0, p1, W1, b1, W2, b2, W3, b3, g1, be1)


def _conv2_head_tc(h, q0, q1, W4, b4, g2, be2, eps2, Wl1, bl1, Wl2, bl2):
    def body(eps_ref, h_ref, q0_ref, q1_ref, w4_ref, b4_ref, g_ref, be_ref,
             wl1_ref, bl1_ref, wl2_ref, bl2_ref, z_ref):
        t = (1.0 + eps_ref[0]) * h_ref[...] + q0_ref[...] + q1_ref[...]
        o = jnp.maximum(jnp.dot(t, w4_ref[...], preferred_element_type=jnp.float32) + b4_ref[...], 0.0)
        m = jnp.sum(o, axis=0, keepdims=True) * (1.0 / _N)
        q = jnp.sum(o * o, axis=0, keepdims=True) * (1.0 / _N)
        rstd = lax.rsqrt(q - m * m + 1e-5)
        hn = (o - m) * rstd * g_ref[...] + be_ref[...]
        z1 = jnp.maximum(jnp.dot(hn, wl1_ref[...], preferred_element_type=jnp.float32) + bl1_ref[...], 0.0)
        z_ref[...] = jnp.dot(z1, wl2_ref[...], preferred_element_type=jnp.float32) + bl2_ref[...]

    return pl.pallas_call(
        body,
        in_specs=[
            pl.BlockSpec(memory_space=pltpu.SMEM),
            _vm((_N, _D)), _vm((_N, _D)), _vm((_N, _D)),
            _vm((_D, _D)), _vm((1, _D)), _vm((1, _D)), _vm((1, _D)),
            _vm((_D, _D)), _vm((1, _D)), _vm((_D, _D)), _vm((1, _D)),
        ],
        out_specs=_vm((_N, _D)),
        out_shape=jax.ShapeDtypeStruct((_N, _D), jnp.float32),
    )(eps2, h, q0, q1, W4, b4, g2, be2, Wl1, bl1, Wl2, bl2)


_RF = 2048
_GF = _TE // _RF


def _fc_tc(w, Wf, bf):
    def body(w_ref, wf_ref, bf_ref, o_ref):
        o_ref[...] = jnp.dot(w_ref[...], wf_ref[...], preferred_element_type=jnp.float32) + bf_ref[...]

    return pl.pallas_call(
        body,
        grid=(_GF,),
        in_specs=[
            pl.BlockSpec((_RF, _D), lambda i: (i, 0)),
            pl.BlockSpec((_D, _C), lambda i: (0, 0)),
            pl.BlockSpec((1, _C), lambda i: (0, 0)),
        ],
        out_specs=pl.BlockSpec((_RF, _C), lambda i: (i, 0)),
        out_shape=jax.ShapeDtypeStruct((_TE, _C), jnp.float32),
    )(w, Wf, bf)


def kernel(x, edge_index, train_edge_id, W1, b1, W2, b2, W3, b3, g1, be1, eps1,
           W4, b4, g2, be2, eps2, Wl1, bl1, Wl2, bl2, Wf, bf):
    src2 = edge_index[0].reshape(_NCHUNK, _K)
    dst2 = edge_index[1].reshape(_NCHUNK, _K)
    sd = jnp.stack([src2, dst2], axis=1)
    te2 = train_edge_id.reshape(_TE // _K, _K)

    b1r, b2r, b3r, b4r = (v.reshape(1, _D) for v in (b1, b2, b3, b4))
    bl1r, bl2r = bl1.reshape(1, _D), bl2.reshape(1, _D)
    g1r, be1r = g1.reshape(1, _D), be1.reshape(1, _D)
    g2r, be2r = g2.reshape(1, _D), be2.reshape(1, _D)
    bfr = bf.reshape(1, _C)
    e1 = eps1.reshape(1)
    e2 = eps2.reshape(1)

    pp = _seg_sum_sc(x, sd)
    h = _mlp1_tc(x, pp[0], pp[1], W1, b1r, W2, b2r, W3, b3r, g1r, be1r, e1)
    qq = _seg_sum_sc(h, sd)
    z = _conv2_head_tc(h, qq[0], qq[1], W4, b4r, g2r, be2r, e2,
                       Wl1, bl1r, Wl2, bl2r)
    w = _readout_sc(z, edge_index[0], edge_index[1], te2)
    return _fc_tc(w, Wf, bfr)
